# baseline (speedup 1.0000x reference)
import jax
import jax.numpy as jnp
from jax import lax
from jax.experimental import pallas as pl
from jax.experimental.pallas import tpu as pltpu
from jax.experimental.pallas import tpu_sc as plsc

N = 10000
NP = 10240
E = 320000
D = 128
HD = D // 2
NS = 16
NC = 2
CH = 128
NCHUNK = 160
EPAD = NS * NCHUNK * CH
RPT = NP // NS
CW = 16
NBUF = 4

_f32 = jnp.float32


def _dotT(x, w):
    return lax.dot_general(x, w, (((1,), (1,)), ((), ())),
                           preferred_element_type=_f32,
                           precision=lax.Precision.HIGHEST)


def _make_sc_segsum(with_counts):
    mesh = plsc.VectorSubcoreMesh(core_axis_name="c", subcore_axis_name="s",
                                  num_cores=NC, num_subcores=NS)
    out_type = [jax.ShapeDtypeStruct((NP, HD), _f32) for _ in range(4)]
    scratch = [
        pltpu.VMEM((NCHUNK, CH), jnp.int32),
        pltpu.VMEM((NCHUNK, CH), jnp.int32),
        [pltpu.VMEM((CH, HD), _f32)] * NBUF,
        pltpu.VMEM_SHARED((NP, HD), _f32),
        [pltpu.SemaphoreType.DMA] * NBUF,
        [pltpu.SemaphoreType.DMA] * NBUF,
    ]
    if with_counts:
        out_type += [jax.ShapeDtypeStruct((NP, CW), _f32),
                     jax.ShapeDtypeStruct((NP, CW), _f32)]
        scratch += [
            pltpu.VMEM((CH, CW), _f32),
            pltpu.VMEM_SHARED((NP, CW), _f32),
            [pltpu.SemaphoreType.DMA] * NBUF,
        ]

    def body(y_ui_lo, y_ui_hi, y_iu_lo, y_iu_hi,
             src_ui, dst_ui, src_iu, dst_iu, zrow, zcnt, ones, *refs):
        if with_counts:
            (a_ui_lo, a_ui_hi, a_iu_lo, a_iu_hi, cnt_ui, cnt_iu,
             src_v, dst_v, bufs, acc_sh, gsem, ssem,
             ones_v, cnt_sh, csem) = refs
        else:
            (a_ui_lo, a_ui_hi, a_iu_lo, a_iu_hi,
             src_v, dst_v, bufs, acc_sh, gsem, ssem) = refs
            cnt_ui = cnt_iu = ones_v = cnt_sh = csem = None

        cid = lax.axis_index("c")
        sid = lax.axis_index("s")

        def half_pass(y_hbm, acc_out, cnt_out, rs):
            pltpu.sync_copy(zrow, acc_sh.at[pl.ds(rs, RPT)])
            do_cnt = cnt_out is not None
            if do_cnt:
                pltpu.sync_copy(zcnt, cnt_sh.at[pl.ds(rs, RPT)])
            plsc.subcore_barrier()

            def gstart(c, b):
                pltpu.async_copy(y_hbm.at[src_v.at[c]], bufs[b], gsem[b])

            def gwait(c, b):
                pltpu.make_async_copy(y_hbm.at[src_v.at[c]], bufs[b],
                                      gsem[b]).wait()

            def sstart(c, b):
                pltpu.async_copy(bufs[b], acc_sh.at[dst_v.at[c]], ssem[b],
                                 add=True)
                if do_cnt:
                    pltpu.async_copy(ones_v, cnt_sh.at[dst_v.at[c]], csem[b],
                                     add=True)

            def swait(c, b):
                pltpu.make_async_copy(bufs[b], acc_sh.at[dst_v.at[c]],
                                      ssem[b]).wait()
                if do_cnt:
                    pltpu.make_async_copy(ones_v, cnt_sh.at[dst_v.at[c]],
                                          csem[b]).wait()

            gstart(0, 0)
            gstart(1, 1)
            gstart(2, 2)
            gwait(0, 0)
            sstart(0, 0)
            gstart(3, 3)
            gwait(1, 1)
            sstart(1, 1)

            def step(i, _):
                for k in range(NBUF):
                    c = 2 + i * NBUF + k
                    b = (2 + k) % NBUF
                    b2 = k
                    swait(c - 2, b2)
                    gstart(c + 2, b2)
                    gwait(c, b)
                    sstart(c, b)
                return 0

            lax.fori_loop(0, (NCHUNK - 4) // NBUF, step, 0)

            for c in (NCHUNK - 2, NCHUNK - 1):
                b = c % NBUF
                swait(c - 2, (c + 2) % NBUF)
                gwait(c, b)
                sstart(c, b)
            for c in (NCHUNK - 2, NCHUNK - 1):
                swait(c, c % NBUF)
            plsc.subcore_barrier()

            pltpu.sync_copy(acc_sh.at[pl.ds(rs, RPT)],
                            acc_out.at[pl.ds(rs, RPT)])
            if do_cnt:
                pltpu.sync_copy(cnt_sh.at[pl.ds(rs, RPT)],
                                cnt_out.at[pl.ds(rs, RPT)])

        def half_pass_sync(y_hbm, acc_out, cnt_out, rs):
            pltpu.sync_copy(zrow, acc_sh.at[pl.ds(rs, RPT)])
            do_cnt = cnt_out is not None
            if do_cnt:
                pltpu.sync_copy(zcnt, cnt_sh.at[pl.ds(rs, RPT)])
            plsc.subcore_barrier()

            pltpu.async_copy(y_hbm.at[src_v.at[0]], bufs[0], gsem[0])

            def step(i, _):
                c0 = i * 2
                pltpu.async_copy(y_hbm.at[src_v.at[c0 + 1]], bufs[1], gsem[1])
                pltpu.make_async_copy(y_hbm.at[src_v.at[c0]], bufs[0],
                                      gsem[0]).wait()
                pltpu.sync_copy(bufs[0], acc_sh.at[dst_v.at[c0]], add=True)
                if do_cnt:
                    pltpu.sync_copy(ones_v, cnt_sh.at[dst_v.at[c0]], add=True)

                @pl.when(c0 + 2 < NCHUNK)
                def _():
                    pltpu.async_copy(y_hbm.at[src_v.at[c0 + 2]], bufs[0],
                                     gsem[0])

                pltpu.make_async_copy(y_hbm.at[src_v.at[c0 + 1]], bufs[1],
                                      gsem[1]).wait()
                pltpu.sync_copy(bufs[1], acc_sh.at[dst_v.at[c0 + 1]],
                                add=True)
                if do_cnt:
                    pltpu.sync_copy(ones_v, cnt_sh.at[dst_v.at[c0 + 1]],
                                    add=True)
                return 0

            lax.fori_loop(0, NCHUNK // 2, step, 0)
            plsc.subcore_barrier()

            pltpu.sync_copy(acc_sh.at[pl.ds(rs, RPT)],
                            acc_out.at[pl.ds(rs, RPT)])
            if do_cnt:
                pltpu.sync_copy(cnt_sh.at[pl.ds(rs, RPT)],
                                cnt_out.at[pl.ds(rs, RPT)])

        def run(src_hbm, dst_hbm, y_lo, y_hi, out_lo, out_hi, cnt_out):
            rs = sid * RPT
            pltpu.sync_copy(src_hbm.at[sid], src_v)
            pltpu.sync_copy(dst_hbm.at[sid], dst_v)
            if cnt_out is not None:
                pltpu.sync_copy(ones, ones_v)
            half_pass_sync(y_lo, out_lo, cnt_out, rs)
            half_pass_sync(y_hi, out_hi, None, rs)

        @pl.when(cid == 0)
        def _():
            run(src_ui, dst_ui, y_ui_lo, y_ui_hi, a_ui_lo, a_ui_hi, cnt_ui)

        @pl.when(cid == 1)
        def _():
            run(src_iu, dst_iu, y_iu_lo, y_iu_hi, a_iu_lo, a_iu_hi, cnt_iu)

    return pl.kernel(body, out_type=out_type, mesh=mesh,
                     scratch_types=scratch,
                     compiler_params=pltpu.CompilerParams(
                         use_tc_tiling_on_sc=False))


_sc_segsum_cnt = _make_sc_segsum(True)
_sc_segsum = _make_sc_segsum(False)


_BR = 2000
_GRID = N // _BR

_row_spec = pl.BlockSpec((_BR, D), lambda i: (i, 0))
_half_spec = pl.BlockSpec((_BR, HD), lambda i: (i, 0))
_cnt_spec = pl.BlockSpec((_BR, CW), lambda i: (i, 0))
_w_spec = pl.BlockSpec((D, D), lambda i: (0, 0))
_b_spec = pl.BlockSpec((1, D), lambda i: (0, 0))

_half_out = jax.ShapeDtypeStruct((N, HD), _f32)
_full_out = jax.ShapeDtypeStruct((N, D), _f32)


def _pre_body(xu, w_ui, xi, w_iu, yu_lo, yu_hi, yi_lo, yi_hi):
    yu = _dotT(xu[...], w_ui[...])
    yi = _dotT(xi[...], w_iu[...])
    yu_lo[...] = yu[:, :HD]
    yu_hi[...] = yu[:, HD:]
    yi_lo[...] = yi[:, :HD]
    yi_hi[...] = yi[:, HD:]


_tc_pre = pl.pallas_call(
    _pre_body,
    grid=(_GRID,),
    in_specs=[_row_spec, _w_spec, _row_spec, _w_spec],
    out_specs=[_half_spec] * 4,
    out_shape=[_half_out] * 4,
)


def _combine(a_lo, a_hi, cnt, b, x_dst, w_r):
    acc = jnp.concatenate([a_lo, a_hi], axis=1)
    inv = 1.0 / jnp.maximum(cnt[:, 0:1], 1.0)
    return jnp.maximum(acc * inv + b + _dotT(x_dst, w_r), 0.0)


def _mid_body(a_ui_lo, a_ui_hi, cnt_ui, b_ui, xi, w_ui_r,
              a_iu_lo, a_iu_hi, cnt_iu, b_iu, xu, w_iu_r,
              w2_ui_l, w2_iu_l,
              item1, user1, y2u_lo, y2u_hi, y2i_lo, y2i_hi):
    it1 = _combine(a_ui_lo[...], a_ui_hi[...], cnt_ui[...], b_ui[...],
                   xi[...], w_ui_r[...])
    us1 = _combine(a_iu_lo[...], a_iu_hi[...], cnt_iu[...], b_iu[...],
                   xu[...], w_iu_r[...])
    item1[...] = it1
    user1[...] = us1
    y2u = _dotT(us1, w2_ui_l[...])
    y2i = _dotT(it1, w2_iu_l[...])
    y2u_lo[...] = y2u[:, :HD]
    y2u_hi[...] = y2u[:, HD:]
    y2i_lo[...] = y2i[:, :HD]
    y2i_hi[...] = y2i[:, HD:]


_tc_mid = pl.pallas_call(
    _mid_body,
    grid=(_GRID,),
    in_specs=[_half_spec, _half_spec, _cnt_spec, _b_spec, _row_spec, _w_spec,
              _half_spec, _half_spec, _cnt_spec, _b_spec, _row_spec, _w_spec,
              _w_spec, _w_spec],
    out_specs=[_row_spec, _row_spec] + [_half_spec] * 4,
    out_shape=[_full_out, _full_out] + [_half_out] * 4,
)


def _post_body(a_ui_lo, a_ui_hi, cnt_ui, b_ui, item1, w_ui_r,
               a_iu_lo, a_iu_hi, cnt_iu, b_iu, user1, w_iu_r,
               item2, user2):
    item2[...] = _combine(a_ui_lo[...], a_ui_hi[...], cnt_ui[...], b_ui[...],
                          item1[...], w_ui_r[...])
    user2[...] = _combine(a_iu_lo[...], a_iu_hi[...], cnt_iu[...], b_iu[...],
                          user1[...], w_iu_r[...])


_tc_post = pl.pallas_call(
    _post_body,
    grid=(_GRID,),
    in_specs=[_half_spec, _half_spec, _cnt_spec, _b_spec, _row_spec, _w_spec,
              _half_spec, _half_spec, _cnt_spec, _b_spec, _row_spec, _w_spec],
    out_specs=[_row_spec, _row_spec],
    out_shape=[_full_out, _full_out],
)


def kernel(x_user, x_item, edge_index_ui, edge_index_iu,
           W1_ui_l, b1_ui_l, W1_ui_r, W1_iu_l, b1_iu_l, W1_iu_r,
           W2_ui_l, b2_ui_l, W2_ui_r, W2_iu_l, b2_iu_l, W2_iu_r):
    pad_src = jnp.zeros((EPAD - E,), jnp.int32)
    pad_dst = jnp.full((EPAD - E,), N, jnp.int32)

    def _tile(idx, pad):
        return jnp.concatenate([idx, pad]).reshape(NS, NCHUNK, CH)

    src_ui = _tile(edge_index_ui[0], pad_src)
    dst_ui = _tile(edge_index_ui[1], pad_dst)
    src_iu = _tile(edge_index_iu[0], pad_src)
    dst_iu = _tile(edge_index_iu[1], pad_dst)

    zrow = jnp.zeros((RPT, HD), _f32)
    zcnt = jnp.zeros((RPT, CW), _f32)
    ones = jnp.ones((CH, CW), _f32)
    b1_ui = b1_ui_l.reshape(1, D)
    b1_iu = b1_iu_l.reshape(1, D)
    b2_ui = b2_ui_l.reshape(1, D)
    b2_iu = b2_iu_l.reshape(1, D)

    yu_lo, yu_hi, yi_lo, yi_hi = _tc_pre(x_user, W1_ui_l, x_item, W1_iu_l)
    a_ui_lo, a_ui_hi, a_iu_lo, a_iu_hi, cnt_ui, cnt_iu = _sc_segsum_cnt(
        yu_lo, yu_hi, yi_lo, yi_hi,
        src_ui, dst_ui, src_iu, dst_iu, zrow, zcnt, ones)
    item1, user1, y2u_lo, y2u_hi, y2i_lo, y2i_hi = _tc_mid(
        a_ui_lo, a_ui_hi, cnt_ui, b1_ui, x_item, W1_ui_r,
        a_iu_lo, a_iu_hi, cnt_iu, b1_iu, x_user, W1_iu_r,
        W2_ui_l, W2_iu_l)

    a2_ui_lo, a2_ui_hi, a2_iu_lo, a2_iu_hi = _sc_segsum(
        y2u_lo, y2u_hi, y2i_lo, y2i_hi,
        src_ui, dst_ui, src_iu, dst_iu, zrow, zcnt, ones)
    item2, user2 = _tc_post(
        a2_ui_lo, a2_ui_hi, cnt_ui, b2_ui, item1, W2_ui_r,
        a2_iu_lo, a2_iu_hi, cnt_iu, b2_iu, user1, W2_iu_r)

    return (user2, item2)


# speedup vs baseline: 1.6700x; 1.6700x over previous
import jax
import jax.numpy as jnp
from jax import lax
from jax.experimental import pallas as pl
from jax.experimental.pallas import tpu as pltpu
from jax.experimental.pallas import tpu_sc as plsc

N = 10000
NP = 10240
E = 320000
D = 128
HD = D // 2
NS = 16
NC = 2
CH = 80
NCHUNK = 252
EPAD = NS * NCHUNK * CH
RPT = NP // NS
CW = 16
NBUF = 4

_f32 = jnp.float32


def _dotT(x, w):
    return lax.dot_general(x, w, (((1,), (1,)), ((), ())),
                           preferred_element_type=_f32,
                           precision=lax.Precision.HIGHEST)


def _make_sc_segsum(with_counts):
    mesh = plsc.VectorSubcoreMesh(core_axis_name="c", subcore_axis_name="s",
                                  num_cores=NC, num_subcores=NS)
    out_type = [jax.ShapeDtypeStruct((NP, HD), _f32) for _ in range(4)]
    scratch = [
        pltpu.VMEM((NCHUNK, CH), jnp.int32),
        pltpu.VMEM((NCHUNK, CH), jnp.int32),
        [pltpu.VMEM((CH, HD), _f32)] * NBUF,
        pltpu.VMEM_SHARED((NP, HD), _f32),
        [pltpu.SemaphoreType.DMA] * NBUF,
        [pltpu.SemaphoreType.DMA] * NBUF,
    ]
    if with_counts:
        out_type += [jax.ShapeDtypeStruct((NP, CW), _f32),
                     jax.ShapeDtypeStruct((NP, CW), _f32)]
        scratch += [
            pltpu.VMEM((CH, CW), _f32),
            pltpu.VMEM_SHARED((NP, CW), _f32),
            [pltpu.SemaphoreType.DMA] * NBUF,
        ]

    def body(y_ui_lo, y_ui_hi, y_iu_lo, y_iu_hi,
             src_ui, dst_ui, src_iu, dst_iu, zrow, zcnt, ones, *refs):
        if with_counts:
            (a_ui_lo, a_ui_hi, a_iu_lo, a_iu_hi, cnt_ui, cnt_iu,
             src_v, dst_v, bufs, acc_sh, gsem, ssem,
             ones_v, cnt_sh, csem) = refs
        else:
            (a_ui_lo, a_ui_hi, a_iu_lo, a_iu_hi,
             src_v, dst_v, bufs, acc_sh, gsem, ssem) = refs
            cnt_ui = cnt_iu = ones_v = cnt_sh = csem = None

        cid = lax.axis_index("c")
        sid = lax.axis_index("s")

        def half_pass(y_hbm, acc_out, cnt_out, rs):
            pltpu.sync_copy(zrow, acc_sh.at[pl.ds(rs, RPT)])
            do_cnt = cnt_out is not None
            if do_cnt:
                pltpu.sync_copy(zcnt, cnt_sh.at[pl.ds(rs, RPT)])
            plsc.subcore_barrier()

            def gstart(c, b):
                pltpu.async_copy(y_hbm.at[src_v.at[c]], bufs[b], gsem[b])

            def gwait(c, b):
                pltpu.make_async_copy(y_hbm.at[src_v.at[c]], bufs[b],
                                      gsem[b]).wait()

            def sstart(c, b):
                pltpu.async_copy(bufs[b], acc_sh.at[dst_v.at[c]], ssem[b],
                                 add=True)
                if do_cnt:
                    pltpu.async_copy(ones_v, cnt_sh.at[dst_v.at[c]], csem[b],
                                     add=True)

            def swait(c, b):
                pltpu.make_async_copy(bufs[b], acc_sh.at[dst_v.at[c]],
                                      ssem[b]).wait()
                if do_cnt:
                    pltpu.make_async_copy(ones_v, cnt_sh.at[dst_v.at[c]],
                                          csem[b]).wait()

            gstart(0, 0)
            gstart(1, 1)
            gstart(2, 2)
            gwait(0, 0)
            sstart(0, 0)
            gstart(3, 3)
            gwait(1, 1)
            sstart(1, 1)

            def step(i, _):
                for k in range(NBUF):
                    c = 2 + i * NBUF + k
                    b = (2 + k) % NBUF
                    b2 = k
                    swait(c - 2, b2)
                    gstart(c + 2, b2)
                    gwait(c, b)
                    sstart(c, b)
                return 0

            lax.fori_loop(0, (NCHUNK - 4) // NBUF, step, 0)

            for c in (NCHUNK - 2, NCHUNK - 1):
                b = c % NBUF
                swait(c - 2, (c + 2) % NBUF)
                gwait(c, b)
                sstart(c, b)
            for c in (NCHUNK - 2, NCHUNK - 1):
                swait(c, c % NBUF)
            plsc.subcore_barrier()

            pltpu.sync_copy(acc_sh.at[pl.ds(rs, RPT)],
                            acc_out.at[pl.ds(rs, RPT)])
            if do_cnt:
                pltpu.sync_copy(cnt_sh.at[pl.ds(rs, RPT)],
                                cnt_out.at[pl.ds(rs, RPT)])

        def half_pass_sync(y_hbm, acc_out, cnt_out, rs):
            pltpu.sync_copy(zrow, acc_sh.at[pl.ds(rs, RPT)])
            do_cnt = cnt_out is not None
            if do_cnt:
                pltpu.sync_copy(zcnt, cnt_sh.at[pl.ds(rs, RPT)])
            plsc.subcore_barrier()

            pltpu.async_copy(y_hbm.at[src_v.at[0]], bufs[0], gsem[0])

            def step(i, _):
                c0 = i * 2
                pltpu.async_copy(y_hbm.at[src_v.at[c0 + 1]], bufs[1], gsem[1])
                pltpu.make_async_copy(y_hbm.at[src_v.at[c0]], bufs[0],
                                      gsem[0]).wait()
                pltpu.sync_copy(bufs[0], acc_sh.at[dst_v.at[c0]], add=True)
                if do_cnt:
                    pltpu.sync_copy(ones_v, cnt_sh.at[dst_v.at[c0]], add=True)

                @pl.when(c0 + 2 < NCHUNK)
                def _():
                    pltpu.async_copy(y_hbm.at[src_v.at[c0 + 2]], bufs[0],
                                     gsem[0])

                pltpu.make_async_copy(y_hbm.at[src_v.at[c0 + 1]], bufs[1],
                                      gsem[1]).wait()
                pltpu.sync_copy(bufs[1], acc_sh.at[dst_v.at[c0 + 1]],
                                add=True)
                if do_cnt:
                    pltpu.sync_copy(ones_v, cnt_sh.at[dst_v.at[c0 + 1]],
                                    add=True)
                return 0

            lax.fori_loop(0, NCHUNK // 2, step, 0)
            plsc.subcore_barrier()

            pltpu.sync_copy(acc_sh.at[pl.ds(rs, RPT)],
                            acc_out.at[pl.ds(rs, RPT)])
            if do_cnt:
                pltpu.sync_copy(cnt_sh.at[pl.ds(rs, RPT)],
                                cnt_out.at[pl.ds(rs, RPT)])

        def run(src_hbm, dst_hbm, y_lo, y_hi, out_lo, out_hi, cnt_out):
            rs = sid * RPT
            pltpu.sync_copy(src_hbm.at[sid], src_v)
            pltpu.sync_copy(dst_hbm.at[sid], dst_v)
            if cnt_out is not None:
                pltpu.sync_copy(ones, ones_v)
            half_pass(y_lo, out_lo, cnt_out, rs)
            half_pass(y_hi, out_hi, None, rs)

        @pl.when(cid == 0)
        def _():
            run(src_ui, dst_ui, y_ui_lo, y_ui_hi, a_ui_lo, a_ui_hi, cnt_ui)

        @pl.when(cid == 1)
        def _():
            run(src_iu, dst_iu, y_iu_lo, y_iu_hi, a_iu_lo, a_iu_hi, cnt_iu)

    return pl.kernel(body, out_type=out_type, mesh=mesh,
                     scratch_types=scratch,
                     compiler_params=pltpu.CompilerParams(
                         use_tc_tiling_on_sc=False))


_sc_segsum_cnt = _make_sc_segsum(True)
_sc_segsum = _make_sc_segsum(False)


_BR = 2000
_GRID = N // _BR

_row_spec = pl.BlockSpec((_BR, D), lambda i: (i, 0))
_half_spec = pl.BlockSpec((_BR, HD), lambda i: (i, 0))
_cnt_spec = pl.BlockSpec((_BR, CW), lambda i: (i, 0))
_w_spec = pl.BlockSpec((D, D), lambda i: (0, 0))
_b_spec = pl.BlockSpec((1, D), lambda i: (0, 0))

_half_out = jax.ShapeDtypeStruct((N, HD), _f32)
_full_out = jax.ShapeDtypeStruct((N, D), _f32)


def _pre_body(xu, w_ui, xi, w_iu, yu_lo, yu_hi, yi_lo, yi_hi):
    yu = _dotT(xu[...], w_ui[...])
    yi = _dotT(xi[...], w_iu[...])
    yu_lo[...] = yu[:, :HD]
    yu_hi[...] = yu[:, HD:]
    yi_lo[...] = yi[:, :HD]
    yi_hi[...] = yi[:, HD:]


_tc_pre = pl.pallas_call(
    _pre_body,
    grid=(_GRID,),
    in_specs=[_row_spec, _w_spec, _row_spec, _w_spec],
    out_specs=[_half_spec] * 4,
    out_shape=[_half_out] * 4,
)


def _combine(a_lo, a_hi, cnt, b, x_dst, w_r):
    acc = jnp.concatenate([a_lo, a_hi], axis=1)
    inv = 1.0 / jnp.maximum(cnt[:, 0:1], 1.0)
    return jnp.maximum(acc * inv + b + _dotT(x_dst, w_r), 0.0)


def _mid_body(a_ui_lo, a_ui_hi, cnt_ui, b_ui, xi, w_ui_r,
              a_iu_lo, a_iu_hi, cnt_iu, b_iu, xu, w_iu_r,
              w2_ui_l, w2_iu_l,
              item1, user1, y2u_lo, y2u_hi, y2i_lo, y2i_hi):
    it1 = _combine(a_ui_lo[...], a_ui_hi[...], cnt_ui[...], b_ui[...],
                   xi[...], w_ui_r[...])
    us1 = _combine(a_iu_lo[...], a_iu_hi[...], cnt_iu[...], b_iu[...],
                   xu[...], w_iu_r[...])
    item1[...] = it1
    user1[...] = us1
    y2u = _dotT(us1, w2_ui_l[...])
    y2i = _dotT(it1, w2_iu_l[...])
    y2u_lo[...] = y2u[:, :HD]
    y2u_hi[...] = y2u[:, HD:]
    y2i_lo[...] = y2i[:, :HD]
    y2i_hi[...] = y2i[:, HD:]


_tc_mid = pl.pallas_call(
    _mid_body,
    grid=(_GRID,),
    in_specs=[_half_spec, _half_spec, _cnt_spec, _b_spec, _row_spec, _w_spec,
              _half_spec, _half_spec, _cnt_spec, _b_spec, _row_spec, _w_spec,
              _w_spec, _w_spec],
    out_specs=[_row_spec, _row_spec] + [_half_spec] * 4,
    out_shape=[_full_out, _full_out] + [_half_out] * 4,
)


def _post_body(a_ui_lo, a_ui_hi, cnt_ui, b_ui, item1, w_ui_r,
               a_iu_lo, a_iu_hi, cnt_iu, b_iu, user1, w_iu_r,
               item2, user2):
    item2[...] = _combine(a_ui_lo[...], a_ui_hi[...], cnt_ui[...], b_ui[...],
                          item1[...], w_ui_r[...])
    user2[...] = _combine(a_iu_lo[...], a_iu_hi[...], cnt_iu[...], b_iu[...],
                          user1[...], w_iu_r[...])


_tc_post = pl.pallas_call(
    _post_body,
    grid=(_GRID,),
    in_specs=[_half_spec, _half_spec, _cnt_spec, _b_spec, _row_spec, _w_spec,
              _half_spec, _half_spec, _cnt_spec, _b_spec, _row_spec, _w_spec],
    out_specs=[_row_spec, _row_spec],
    out_shape=[_full_out, _full_out],
)


def kernel(x_user, x_item, edge_index_ui, edge_index_iu,
           W1_ui_l, b1_ui_l, W1_ui_r, W1_iu_l, b1_iu_l, W1_iu_r,
           W2_ui_l, b2_ui_l, W2_ui_r, W2_iu_l, b2_iu_l, W2_iu_r):
    pad_src = jnp.zeros((EPAD - E,), jnp.int32)
    pad_dst = jnp.full((EPAD - E,), N, jnp.int32)

    def _tile(idx, pad):
        return jnp.concatenate([idx, pad]).reshape(NS, NCHUNK, CH)

    src_ui = _tile(edge_index_ui[0], pad_src)
    dst_ui = _tile(edge_index_ui[1], pad_dst)
    src_iu = _tile(edge_index_iu[0], pad_src)
    dst_iu = _tile(edge_index_iu[1], pad_dst)

    zrow = jnp.zeros((RPT, HD), _f32)
    zcnt = jnp.zeros((RPT, CW), _f32)
    ones = jnp.ones((CH, CW), _f32)
    b1_ui = b1_ui_l.reshape(1, D)
    b1_iu = b1_iu_l.reshape(1, D)
    b2_ui = b2_ui_l.reshape(1, D)
    b2_iu = b2_iu_l.reshape(1, D)

    yu_lo, yu_hi, yi_lo, yi_hi = _tc_pre(x_user, W1_ui_l, x_item, W1_iu_l)
    a_ui_lo, a_ui_hi, a_iu_lo, a_iu_hi, cnt_ui, cnt_iu = _sc_segsum_cnt(
        yu_lo, yu_hi, yi_lo, yi_hi,
        src_ui, dst_ui, src_iu, dst_iu, zrow, zcnt, ones)
    item1, user1, y2u_lo, y2u_hi, y2i_lo, y2i_hi = _tc_mid(
        a_ui_lo, a_ui_hi, cnt_ui, b1_ui, x_item, W1_ui_r,
        a_iu_lo, a_iu_hi, cnt_iu, b1_iu, x_user, W1_iu_r,
        W2_ui_l, W2_iu_l)

    a2_ui_lo, a2_ui_hi, a2_iu_lo, a2_iu_hi = _sc_segsum(
        y2u_lo, y2u_hi, y2i_lo, y2i_hi,
        src_ui, dst_ui, src_iu, dst_iu, zrow, zcnt, ones)
    item2, user2 = _tc_post(
        a2_ui_lo, a2_ui_hi, cnt_ui, b2_ui, item1, W2_ui_r,
        a2_iu_lo, a2_iu_hi, cnt_iu, b2_iu, user1, W2_iu_r)

    return (user2, item2)


# speedup vs baseline: 2.8609x; 1.7132x over previous
import jax
import jax.numpy as jnp
from jax import lax
from jax.experimental import pallas as pl
from jax.experimental.pallas import tpu as pltpu
from jax.experimental.pallas import tpu_sc as plsc

N = 10000
NP = 10240
E = 320000
D = 128
NS = 16
NC = 2
CH = 80
NCHUNK = 250
RPT = NP // NS
CW = 16

_f32 = jnp.float32
_bf16 = jnp.bfloat16


def _dotT(x, w):
    return lax.dot_general(x, w, (((1,), (1,)), ((), ())),
                           preferred_element_type=_f32,
                           precision=lax.Precision.HIGHEST)


def _make_sc_segsum(with_counts):
    mesh = plsc.VectorSubcoreMesh(core_axis_name="c", subcore_axis_name="s",
                                  num_cores=NC, num_subcores=NS)
    out_type = [jax.ShapeDtypeStruct((NP, D), _bf16),
                jax.ShapeDtypeStruct((NP, D), _bf16)]
    scratch = [
        pltpu.VMEM((NCHUNK, CH), jnp.int32),
        pltpu.VMEM((NCHUNK, CH), jnp.int32),
        pltpu.VMEM((CH, D), _bf16),
        pltpu.VMEM((CH, D), _bf16),
        pltpu.VMEM_SHARED((NP, D), _bf16),
        pltpu.SemaphoreType.DMA,
        pltpu.SemaphoreType.DMA,
    ]
    if with_counts:
        out_type += [jax.ShapeDtypeStruct((NP, CW), _f32),
                     jax.ShapeDtypeStruct((NP, CW), _f32)]
        scratch += [
            pltpu.VMEM((CH, CW), _f32),
            pltpu.VMEM_SHARED((NP, CW), _f32),
        ]

    def body(y_ui, y_iu, src_ui, dst_ui, src_iu, dst_iu, zrow, zcnt, ones,
             *refs):
        if with_counts:
            (acc_ui, acc_iu, cnt_ui, cnt_iu,
             src_v, dst_v, buf0, buf1, acc_sh, sem0, sem1,
             ones_v, cnt_sh) = refs
        else:
            (acc_ui, acc_iu,
             src_v, dst_v, buf0, buf1, acc_sh, sem0, sem1) = refs
            cnt_ui = cnt_iu = ones_v = cnt_sh = None

        cid = lax.axis_index("c")
        sid = lax.axis_index("s")

        def run(src_hbm, dst_hbm, y_hbm, acc_out, cnt_out):
            rs = sid * RPT
            do_cnt = cnt_out is not None
            pltpu.sync_copy(src_hbm.at[sid], src_v)
            pltpu.sync_copy(dst_hbm.at[sid], dst_v)
            pltpu.sync_copy(zrow, acc_sh.at[pl.ds(rs, RPT)])
            if do_cnt:
                pltpu.sync_copy(ones, ones_v)
                pltpu.sync_copy(zcnt, cnt_sh.at[pl.ds(rs, RPT)])
            plsc.subcore_barrier()

            pltpu.async_copy(y_hbm.at[src_v.at[0]], buf0, sem0)

            def step(i, _):
                c0 = i * 2
                pltpu.async_copy(y_hbm.at[src_v.at[c0 + 1]], buf1, sem1)
                pltpu.make_async_copy(y_hbm.at[src_v.at[c0]], buf0,
                                      sem0).wait()
                pltpu.sync_copy(buf0, acc_sh.at[dst_v.at[c0]], add=True)
                if do_cnt:
                    pltpu.sync_copy(ones_v, cnt_sh.at[dst_v.at[c0]], add=True)

                @pl.when(c0 + 2 < NCHUNK)
                def _():
                    pltpu.async_copy(y_hbm.at[src_v.at[c0 + 2]], buf0, sem0)

                pltpu.make_async_copy(y_hbm.at[src_v.at[c0 + 1]], buf1,
                                      sem1).wait()
                pltpu.sync_copy(buf1, acc_sh.at[dst_v.at[c0 + 1]], add=True)
                if do_cnt:
                    pltpu.sync_copy(ones_v, cnt_sh.at[dst_v.at[c0 + 1]],
                                    add=True)
                return 0

            lax.fori_loop(0, NCHUNK // 2, step, 0)
            plsc.subcore_barrier()

            pltpu.sync_copy(acc_sh.at[pl.ds(rs, RPT)],
                            acc_out.at[pl.ds(rs, RPT)])
            if do_cnt:
                pltpu.sync_copy(cnt_sh.at[pl.ds(rs, RPT)],
                                cnt_out.at[pl.ds(rs, RPT)])

        @pl.when(cid == 0)
        def _():
            run(src_ui, dst_ui, y_ui, acc_ui, cnt_ui)

        @pl.when(cid == 1)
        def _():
            run(src_iu, dst_iu, y_iu, acc_iu, cnt_iu)

    return pl.kernel(body, out_type=out_type, mesh=mesh,
                     scratch_types=scratch,
                     compiler_params=pltpu.CompilerParams(
                         use_tc_tiling_on_sc=False))


_sc_segsum_cnt = _make_sc_segsum(True)
_sc_segsum = _make_sc_segsum(False)


_BR = 2000
_GRID = N // _BR

_row_spec = pl.BlockSpec((_BR, D), lambda i: (i, 0))
_cnt_spec = pl.BlockSpec((_BR, CW), lambda i: (i, 0))
_w_spec = pl.BlockSpec((D, D), lambda i: (0, 0))
_b_spec = pl.BlockSpec((1, D), lambda i: (0, 0))

_full_f32 = jax.ShapeDtypeStruct((N, D), _f32)
_full_bf16 = jax.ShapeDtypeStruct((N, D), _bf16)


def _pre_body(xu, w_ui, xi, w_iu, yu, yi):
    yu[...] = _dotT(xu[...], w_ui[...]).astype(_bf16)
    yi[...] = _dotT(xi[...], w_iu[...]).astype(_bf16)


_tc_pre = pl.pallas_call(
    _pre_body,
    grid=(_GRID,),
    in_specs=[_row_spec, _w_spec, _row_spec, _w_spec],
    out_specs=[_row_spec, _row_spec],
    out_shape=[_full_bf16, _full_bf16],
)


def _combine(acc, cnt, b, x_dst, w_r):
    inv = 1.0 / jnp.maximum(cnt[:, 0:1], 1.0)
    return jnp.maximum(acc.astype(_f32) * inv + b + _dotT(x_dst, w_r), 0.0)


def _mid_body(a_ui, cnt_ui, b_ui, xi, w_ui_r,
              a_iu, cnt_iu, b_iu, xu, w_iu_r,
              w2_ui_l, w2_iu_l,
              item1, user1, y2u, y2i):
    it1 = _combine(a_ui[...], cnt_ui[...], b_ui[...], xi[...], w_ui_r[...])
    us1 = _combine(a_iu[...], cnt_iu[...], b_iu[...], xu[...], w_iu_r[...])
    item1[...] = it1
    user1[...] = us1
    y2u[...] = _dotT(us1, w2_ui_l[...]).astype(_bf16)
    y2i[...] = _dotT(it1, w2_iu_l[...]).astype(_bf16)


_tc_mid = pl.pallas_call(
    _mid_body,
    grid=(_GRID,),
    in_specs=[_row_spec, _cnt_spec, _b_spec, _row_spec, _w_spec,
              _row_spec, _cnt_spec, _b_spec, _row_spec, _w_spec,
              _w_spec, _w_spec],
    out_specs=[_row_spec] * 4,
    out_shape=[_full_f32, _full_f32, _full_bf16, _full_bf16],
)


def _post_body(a_ui, cnt_ui, b_ui, item1, w_ui_r,
               a_iu, cnt_iu, b_iu, user1, w_iu_r,
               item2, user2):
    item2[...] = _combine(a_ui[...], cnt_ui[...], b_ui[...], item1[...],
                          w_ui_r[...])
    user2[...] = _combine(a_iu[...], cnt_iu[...], b_iu[...], user1[...],
                          w_iu_r[...])


_tc_post = pl.pallas_call(
    _post_body,
    grid=(_GRID,),
    in_specs=[_row_spec, _cnt_spec, _b_spec, _row_spec, _w_spec,
              _row_spec, _cnt_spec, _b_spec, _row_spec, _w_spec],
    out_specs=[_row_spec, _row_spec],
    out_shape=[_full_f32, _full_f32],
)


def kernel(x_user, x_item, edge_index_ui, edge_index_iu,
           W1_ui_l, b1_ui_l, W1_ui_r, W1_iu_l, b1_iu_l, W1_iu_r,
           W2_ui_l, b2_ui_l, W2_ui_r, W2_iu_l, b2_iu_l, W2_iu_r):
    src_ui = edge_index_ui[0].reshape(NS, NCHUNK, CH)
    dst_ui = edge_index_ui[1].reshape(NS, NCHUNK, CH)
    src_iu = edge_index_iu[0].reshape(NS, NCHUNK, CH)
    dst_iu = edge_index_iu[1].reshape(NS, NCHUNK, CH)

    zrow = jnp.zeros((RPT, D), _bf16)
    zcnt = jnp.zeros((RPT, CW), _f32)
    ones = jnp.ones((CH, CW), _f32)
    b1_ui = b1_ui_l.reshape(1, D)
    b1_iu = b1_iu_l.reshape(1, D)
    b2_ui = b2_ui_l.reshape(1, D)
    b2_iu = b2_iu_l.reshape(1, D)

    yu, yi = _tc_pre(x_user, W1_ui_l, x_item, W1_iu_l)
    a_ui, a_iu, cnt_ui, cnt_iu = _sc_segsum_cnt(
        yu, yi, src_ui, dst_ui, src_iu, dst_iu, zrow, zcnt, ones)
    item1, user1, y2u, y2i = _tc_mid(
        a_ui, cnt_ui, b1_ui, x_item, W1_ui_r,
        a_iu, cnt_iu, b1_iu, x_user, W1_iu_r,
        W2_ui_l, W2_iu_l)

    a2_ui, a2_iu = _sc_segsum(
        y2u, y2i, src_ui, dst_ui, src_iu, dst_iu, zrow, zcnt, ones)
    item2, user2 = _tc_post(
        a2_ui, cnt_ui, b2_ui, item1, W2_ui_r,
        a2_iu, cnt_iu, b2_iu, user1, W2_iu_r)

    return (user2, item2)
